# baseline (speedup 1.0000x reference)
import jax
import jax.numpy as jnp
from jax.experimental import pallas as pl
from jax.experimental.pallas import tpu as pltpu

B = 32
N = 32
HS = 501
NZ = 56
HSP = 512
NZP = 128

_TNUMS = (((1,), (1,)), ((), ()))


def _dvae_body(X_ref, dc_ref, ash_ref, small_ref, gme_ref,
               whh_raw, wg_raw, wm_raw, wf_raw,
               out_ref, G, whh_s, wgm_s, wf_s):
    bf16 = jnp.bfloat16
    whh_s[...] = jnp.zeros_like(whh_s)
    wgm_s[...] = jnp.zeros_like(wgm_s)
    wf_s[...] = jnp.zeros_like(wf_s)
    whh_s[0:HS, 0:HS] = whh_raw[0:HS, :].astype(bf16)
    whh_s[HSP:HSP + HS, 0:HS] = whh_raw[HS:2 * HS, :].astype(bf16)
    whh_s[2 * HSP:2 * HSP + HS, 0:HS] = whh_raw[2 * HS:3 * HS, :].astype(bf16)
    wgm_s[0:HS, 0:HS] = wg_raw[:, 0:HS].astype(bf16)
    wgm_s[HSP:HSP + HS, 0:HS] = wm_raw[:, 0:HS].astype(bf16)
    wf_s[0:NZ, 0:HS] = wf_raw[...]
    wi3 = small_ref[0:1, :]
    bi3 = small_ref[1:2, :]
    bh3 = small_ref[2:3, :]
    bgm = small_ref[3:4, 0:2 * HSP]
    bfp = small_ref[4:5, 0:NZP]

    G[...] = jnp.zeros_like(G)
    s_iota = jax.lax.broadcasted_iota(jnp.int32, (N, B), 0)

    def step(w, Hprev):
        coef = jnp.where(s_iota < w, ash_ref[pl.ds(w, 1)][0], 0.0)
        stable = jnp.sum(coef[:, :, None] * G[...], axis=0)
        gm = (jax.lax.dot_general(Hprev.astype(bf16), wgm_s[...], _TNUMS,
                                  preferred_element_type=jnp.float32)
              + gme_ref[pl.ds(w, 1)] + bgm)
        gated = jax.nn.sigmoid(gm[:, :HSP]) * gm[:, HSP:]
        G[pl.ds(w, 1)] = gated[None]
        cc = dc_ref[pl.ds(w, 1)][0][:, 0:1]
        Hagg = stable + cc * gated
        xv = X_ref[pl.ds(w, 1)][0][:, 0:1]
        gi = xv * wi3 + bi3
        gh = (jax.lax.dot_general(Hagg.astype(bf16), whh_s[...], _TNUMS,
                                  preferred_element_type=jnp.float32) + bh3)
        r = jax.nn.sigmoid(gi[:, :HSP] + gh[:, :HSP])
        z = jax.nn.sigmoid(gi[:, HSP:2 * HSP] + gh[:, HSP:2 * HSP])
        n = jnp.tanh(gi[:, 2 * HSP:] + r * gh[:, 2 * HSP:])
        return (1.0 - z) * n + z * Hagg

    Hlast = jax.lax.fori_loop(0, N, step, jnp.zeros((B, HSP), jnp.float32))
    out_ref[...] = jax.lax.dot_general(
        Hlast, wf_s[...], _TNUMS, preferred_element_type=jnp.float32) + bfp


def kernel(x, adj, W_ih, W_hh, b_ih, b_hh, Wg, bg, Wm, Wf, bf):
    f32 = jnp.float32

    def _row(a, n3):
        return jnp.pad(a.reshape(-1)[None], ((0, 0), (0, n3 - a.size)))

    def _blocks(vec, nblk):
        return jnp.concatenate(
            [jnp.pad(vec[k * HS:(k + 1) * HS], (0, HSP - HS))
             for k in range(nblk)])[None]

    small = jnp.concatenate([
        _blocks(W_ih[:, 0], 3),
        _blocks(b_ih, 3),
        _blocks(b_hh, 3),
        _row(jnp.pad(bg, (0, HSP - HS)), 3 * HSP),
        _row(jnp.pad(bf, (0, NZP - NZ)), 3 * HSP),
    ], axis=0)
    gme = jnp.concatenate(
        [jnp.pad(Wg[:, HS:].T, ((0, 0), (0, HSP - HS))),
         jnp.pad(Wm[:, HS:].T, ((0, 0), (0, HSP - HS)))], axis=1)
    gme_sh = jnp.concatenate([jnp.zeros((1, 2 * HSP), f32), gme[:N - 1]])
    adj_eff = adj.astype(f32) * jnp.triu(jnp.ones((N, N), f32), k=1)
    a_t = jnp.transpose(adj_eff, (2, 1, 0))
    ash = jnp.concatenate([jnp.zeros((N, 1, B), f32), a_t[:, :N - 1, :]], axis=1)
    dc = jnp.concatenate(
        [jnp.zeros((B, 1), f32),
         jnp.diagonal(adj_eff, offset=1, axis1=1, axis2=2)], axis=1)
    dcb = jnp.broadcast_to(dc.T[:, :, None], (N, B, 128))
    X = jnp.broadcast_to(x.T[:, :, None], (N, B, 128)).astype(f32)

    out = pl.pallas_call(
        _dvae_body,
        out_shape=jax.ShapeDtypeStruct((B, NZP), f32),
        scratch_shapes=[
            pltpu.VMEM((N, B, HSP), f32),
            pltpu.VMEM((3 * HSP, HSP), jnp.bfloat16),
            pltpu.VMEM((2 * HSP, HSP), jnp.bfloat16),
            pltpu.VMEM((NZP, HSP), f32),
        ],
    )(X, dcb, ash, small, gme_sh, W_hh, Wg, Wm, Wf)
    return out[:, :NZ][:, :, None]


# speedup vs baseline: 1.1879x; 1.1879x over previous
import jax
import jax.numpy as jnp
from jax.experimental import pallas as pl
from jax.experimental.pallas import tpu as pltpu

B = 32
N = 32
HS = 501
NZ = 56
HSP = 512
NZP = 128

_TNUMS = (((1,), (1,)), ((), ()))


def _dvae_body(X_ref, dc_ref, ash_ref, small_ref, gme_ref,
               whh_raw, wg_raw, wm_raw, wf_raw,
               out_ref, G, whh_s, wgm_s, wf_s):
    bf16 = jnp.bfloat16
    def _tpadded(blk):
        return jnp.transpose(
            jnp.pad(blk.astype(bf16), ((0, HSP - blk.shape[0]),
                                       (0, HSP - blk.shape[1]))))

    whh_s[:, 0:HSP] = _tpadded(whh_raw[0:HS, :])
    whh_s[:, HSP:2 * HSP] = _tpadded(whh_raw[HS:2 * HS, :])
    whh_s[:, 2 * HSP:3 * HSP] = _tpadded(whh_raw[2 * HS:3 * HS, :])
    wgm_s[:, 0:HSP] = _tpadded(wg_raw[:, 0:HS])
    wgm_s[:, HSP:2 * HSP] = _tpadded(wm_raw[:, 0:HS])
    wf_s[...] = jnp.transpose(
        jnp.pad(wf_raw[...], ((0, NZP - NZ), (0, HSP - HS))))
    wi3 = small_ref[0:1, :]
    bi3 = small_ref[1:2, :]
    bh3 = small_ref[2:3, :]
    bgm = small_ref[3:4, 0:2 * HSP]
    bfp = small_ref[4:5, 0:NZP]

    G[...] = jnp.zeros_like(G)
    s_iota = jax.lax.broadcasted_iota(jnp.int32, (N, B), 0)

    def step(w, Hprev):
        coef = jnp.where(s_iota < w, ash_ref[pl.ds(w, 1)][0], 0.0)
        stable = jnp.sum(coef[:, :, None] * G[...], axis=0)
        gm = (jnp.dot(Hprev.astype(bf16), wgm_s[...],
                      preferred_element_type=jnp.float32)
              + gme_ref[pl.ds(w, 1)] + bgm)
        gated = jax.nn.sigmoid(gm[:, :HSP]) * gm[:, HSP:]
        G[pl.ds(w, 1)] = gated[None]
        cc = dc_ref[pl.ds(w, 1)][0][:, 0:1]
        Hagg = stable + cc * gated
        xv = X_ref[pl.ds(w, 1)][0][:, 0:1]
        gi = xv * wi3 + bi3
        gh = (jnp.dot(Hagg.astype(bf16), whh_s[...],
                      preferred_element_type=jnp.float32) + bh3)
        r = jax.nn.sigmoid(gi[:, :HSP] + gh[:, :HSP])
        z = jax.nn.sigmoid(gi[:, HSP:2 * HSP] + gh[:, HSP:2 * HSP])
        n = jnp.tanh(gi[:, 2 * HSP:] + r * gh[:, 2 * HSP:])
        return (1.0 - z) * n + z * Hagg

    Hlast = jax.lax.fori_loop(0, N, step, jnp.zeros((B, HSP), jnp.float32))
    out_ref[...] = jnp.dot(Hlast, wf_s[...],
                           preferred_element_type=jnp.float32) + bfp


def kernel(x, adj, W_ih, W_hh, b_ih, b_hh, Wg, bg, Wm, Wf, bf):
    f32 = jnp.float32

    def _row(a, n3):
        return jnp.pad(a.reshape(-1)[None], ((0, 0), (0, n3 - a.size)))

    def _blocks(vec, nblk):
        return jnp.concatenate(
            [jnp.pad(vec[k * HS:(k + 1) * HS], (0, HSP - HS))
             for k in range(nblk)])[None]

    small = jnp.concatenate([
        _blocks(W_ih[:, 0], 3),
        _blocks(b_ih, 3),
        _blocks(b_hh, 3),
        _row(jnp.pad(bg, (0, HSP - HS)), 3 * HSP),
        _row(jnp.pad(bf, (0, NZP - NZ)), 3 * HSP),
    ], axis=0)
    gme = jnp.concatenate(
        [jnp.pad(Wg[:, HS:].T, ((0, 0), (0, HSP - HS))),
         jnp.pad(Wm[:, HS:].T, ((0, 0), (0, HSP - HS)))], axis=1)
    gme_sh = jnp.concatenate([jnp.zeros((1, 2 * HSP), f32), gme[:N - 1]])
    adj_eff = adj.astype(f32) * jnp.triu(jnp.ones((N, N), f32), k=1)
    a_t = jnp.transpose(adj_eff, (2, 1, 0))
    ash = jnp.concatenate([jnp.zeros((N, 1, B), f32), a_t[:, :N - 1, :]], axis=1)
    dc = jnp.concatenate(
        [jnp.zeros((B, 1), f32),
         jnp.diagonal(adj_eff, offset=1, axis1=1, axis2=2)], axis=1)
    dcb = jnp.broadcast_to(dc.T[:, :, None], (N, B, 128))
    X = jnp.broadcast_to(x.T[:, :, None], (N, B, 128)).astype(f32)

    out = pl.pallas_call(
        _dvae_body,
        out_shape=jax.ShapeDtypeStruct((B, NZP), f32),
        scratch_shapes=[
            pltpu.VMEM((N, B, HSP), f32),
            pltpu.VMEM((HSP, 3 * HSP), jnp.bfloat16),
            pltpu.VMEM((HSP, 2 * HSP), jnp.bfloat16),
            pltpu.VMEM((HSP, NZP), f32),
        ],
    )(X, dcb, ash, small, gme_sh, W_hh, Wg, Wm, Wf)
    return out[:, :NZ][:, :, None]


# speedup vs baseline: 1.3974x; 1.1763x over previous
import jax
import jax.numpy as jnp
from jax.experimental import pallas as pl
from jax.experimental.pallas import tpu as pltpu

B = 32
N = 32
HS = 501
NZ = 56
HSP = 512
NZP = 128


def _bmul(scal128, vec, nblk):
    return jnp.concatenate(
        [scal128 * vec[:, 128 * k:128 * (k + 1)] for k in range(nblk)], axis=1)


def _dvae_body(x_ref, adj_ref, at_ref, wi_r, bi_r, bh_r, bg_r, bf_r,
               whh_raw, wg_raw, wm_raw, wf_raw, out_ref,
               G, whh_s, wgm_s, wf_s, gme_s, sm_s, Xb_s, dcb_s):
    f32 = jnp.float32
    bf16 = jnp.bfloat16

    def _tpadded(blk):
        return jnp.transpose(
            jnp.pad(blk.astype(bf16), ((0, HSP - blk.shape[0]),
                                       (0, HSP - blk.shape[1]))))

    whh_s[:, 0:HSP] = _tpadded(whh_raw[0:HS, :])
    whh_s[:, HSP:2 * HSP] = _tpadded(whh_raw[HS:2 * HS, :])
    whh_s[:, 2 * HSP:3 * HSP] = _tpadded(whh_raw[2 * HS:3 * HS, :])
    wgm_s[:, 0:HSP] = _tpadded(wg_raw[:, 0:HS])
    wgm_s[:, HSP:2 * HSP] = _tpadded(wm_raw[:, 0:HS])
    wf_s[...] = jnp.transpose(
        jnp.pad(wf_raw[...], ((0, NZP - NZ), (0, HSP - HS))))
    gme_s[:, 0:HSP] = jnp.transpose(jnp.pad(wg_raw[:, HS:], ((0, HSP - HS),
                                                             (0, 0))))
    gme_s[:, HSP:] = jnp.transpose(jnp.pad(wm_raw[:, HS:], ((0, HSP - HS),
                                                            (0, 0))))
    sm_s[...] = jnp.zeros_like(sm_s)
    z11 = jnp.zeros((1, HSP - HS), f32)

    def _blocks(row):
        return jnp.concatenate([row[:, 0:HS], z11, row[:, HS:2 * HS], z11,
                                row[:, 2 * HS:3 * HS], z11], axis=1)

    sm_s[0:1, :] = _blocks(wi_r[...])
    sm_s[1:2, :] = _blocks(bi_r[...])
    sm_s[2:3, :] = _blocks(bh_r[...])
    sm_s[3:4, 0:HS] = bg_r[...]
    sm_s[4:5, 0:NZ] = bf_r[...]
    for w in range(N):
        Xb_s[w] = jnp.broadcast_to(x_ref[:, w:w + 1], (B, 128))
        dcb_s[w] = (jnp.zeros((B, 128), f32) if w == 0 else
                    jnp.broadcast_to(adj_ref[:, w - 1, w:w + 1].astype(f32),
                                     (B, 128)))

    G[...] = jnp.zeros_like(G)
    u_iota = jax.lax.broadcasted_iota(jnp.int32, (N, B), 0)

    def step(w, Hprev):
        wm1 = jnp.maximum(w - 1, 0)
        coef = jnp.where(u_iota < w - 1, at_ref[pl.ds(w, 1)][0], 0.0)
        stable = jnp.sum(coef[:, :, None] * G[...], axis=0)
        gm = (jnp.dot(Hprev.astype(bf16), wgm_s[...],
                      preferred_element_type=f32)
              + gme_s[pl.ds(wm1, 1)] + sm_s[3:4, 0:2 * HSP])
        gated = jax.nn.sigmoid(gm[:, :HSP]) * gm[:, HSP:]
        G[pl.ds(wm1, 1)] = gated[None]
        Hagg = stable + _bmul(dcb_s[pl.ds(w, 1)][0], gated, 4)
        gi = _bmul(Xb_s[pl.ds(w, 1)][0], sm_s[0:1, :], 12) + sm_s[1:2, :]
        gh = (jnp.dot(Hagg.astype(bf16), whh_s[...],
                      preferred_element_type=f32) + sm_s[2:3, :])
        r = jax.nn.sigmoid(gi[:, :HSP] + gh[:, :HSP])
        z = jax.nn.sigmoid(gi[:, HSP:2 * HSP] + gh[:, HSP:2 * HSP])
        n = jnp.tanh(gi[:, 2 * HSP:] + r * gh[:, 2 * HSP:])
        return (1.0 - z) * n + z * Hagg

    Hlast = jax.lax.fori_loop(0, N, step, jnp.zeros((B, HSP), f32))
    out_ref[...] = jnp.dot(Hlast, wf_s[...],
                           preferred_element_type=f32) + sm_s[4:5, 0:NZP]


def kernel(x, adj, W_ih, W_hh, b_ih, b_hh, Wg, bg, Wm, Wf, bf):
    f32 = jnp.float32
    a_t = jnp.transpose(adj, (2, 1, 0)).astype(f32)

    out = pl.pallas_call(
        _dvae_body,
        out_shape=jax.ShapeDtypeStruct((B, NZP), f32),
        scratch_shapes=[
            pltpu.VMEM((N, B, HSP), f32),
            pltpu.VMEM((HSP, 3 * HSP), jnp.bfloat16),
            pltpu.VMEM((HSP, 2 * HSP), jnp.bfloat16),
            pltpu.VMEM((HSP, NZP), f32),
            pltpu.VMEM((N, 2 * HSP), f32),
            pltpu.VMEM((8, 3 * HSP), f32),
            pltpu.VMEM((N, B, 128), f32),
            pltpu.VMEM((N, B, 128), f32),
        ],
    )(x, adj, a_t, W_ih[:, 0][None], b_ih[None], b_hh[None], bg[None],
      bf[None], W_hh, Wg, Wm, Wf)
    return out[:, :NZ][:, :, None]


# speedup vs baseline: 1.4103x; 1.0093x over previous
import jax
import jax.numpy as jnp
from jax.experimental import pallas as pl
from jax.experimental.pallas import tpu as pltpu

B = 32
N = 32
HS = 501
NZ = 56
HSP = 512
NZP = 128


def _bmul(scal128, vec, nblk):
    return jnp.concatenate(
        [scal128 * vec[:, 128 * k:128 * (k + 1)] for k in range(nblk)], axis=1)


def _dvae_body(x_ref, adj_ref, at_ref, wi_r, bi_r, bh_r, bg_r, bf_r,
               whh_raw, wg_raw, wm_raw, wf_raw, out_ref,
               G, whh_s, wgm_s, wf_s, gme_s, sm_s, Xb_s, dcb_s):
    f32 = jnp.float32
    bf16 = jnp.bfloat16

    def _tpadded(blk):
        return jnp.transpose(
            jnp.pad(blk.astype(bf16), ((0, HSP - blk.shape[0]),
                                       (0, HSP - blk.shape[1]))))

    whh_s[:, 0:HSP] = _tpadded(whh_raw[0:HS, :])
    whh_s[:, HSP:2 * HSP] = _tpadded(whh_raw[HS:2 * HS, :])
    whh_s[:, 2 * HSP:3 * HSP] = _tpadded(whh_raw[2 * HS:3 * HS, :])
    wgm_s[:, 0:HSP] = _tpadded(wg_raw[:, 0:HS])
    wgm_s[:, HSP:2 * HSP] = _tpadded(wm_raw[:, 0:HS])
    wf_s[...] = jnp.transpose(
        jnp.pad(wf_raw[...], ((0, NZP - NZ), (0, HSP - HS))))
    gme_s[:, 0:HSP] = jnp.transpose(jnp.pad(wg_raw[:, HS:], ((0, HSP - HS),
                                                             (0, 0))))
    gme_s[:, HSP:] = jnp.transpose(jnp.pad(wm_raw[:, HS:], ((0, HSP - HS),
                                                            (0, 0))))
    sm_s[...] = jnp.zeros_like(sm_s)
    z11 = jnp.zeros((1, HSP - HS), f32)

    def _blocks(row):
        return jnp.concatenate([row[:, 0:HS], z11, row[:, HS:2 * HS], z11,
                                row[:, 2 * HS:3 * HS], z11], axis=1)

    sm_s[0:1, :] = _blocks(wi_r[...])
    sm_s[1:2, :] = _blocks(bi_r[...])
    sm_s[2:3, :] = _blocks(bh_r[...])
    sm_s[3:4, 0:HS] = bg_r[...]
    sm_s[4:5, 0:NZ] = bf_r[...]
    for w in range(N):
        Xb_s[w] = jnp.broadcast_to(x_ref[:, w:w + 1], (B, 128))
        dcb_s[w] = (jnp.zeros((B, 128), f32) if w == 0 else
                    jnp.broadcast_to(adj_ref[:, w - 1, w:w + 1].astype(f32),
                                     (B, 128)))

    G[...] = jnp.zeros_like(G)

    def _mkstep(ns):
        u_iota = jax.lax.broadcasted_iota(jnp.int32, (ns, B), 0)

        def step(w, Hprev):
            wm1 = jnp.maximum(w - 1, 0)
            coef = jnp.where(u_iota < w - 1,
                             at_ref[pl.ds(w, 1)][0][0:ns, :], 0.0)
            stable = jnp.sum(coef[:, :, None] * G[0:ns], axis=0)
            gm = (jnp.dot(Hprev.astype(bf16), wgm_s[...],
                          preferred_element_type=f32)
                  + gme_s[pl.ds(wm1, 1)] + sm_s[3:4, 0:2 * HSP])
            gated = jax.nn.sigmoid(gm[:, :HSP]) * gm[:, HSP:]
            G[pl.ds(wm1, 1)] = gated[None]
            Hagg = stable + _bmul(dcb_s[pl.ds(w, 1)][0], gated, 4)
            gi = _bmul(Xb_s[pl.ds(w, 1)][0], sm_s[0:1, :], 12) + sm_s[1:2, :]
            gh = (jnp.dot(Hagg.astype(bf16), whh_s[...],
                          preferred_element_type=f32) + sm_s[2:3, :])
            r = jax.nn.sigmoid(gi[:, :HSP] + gh[:, :HSP])
            z = jax.nn.sigmoid(gi[:, HSP:2 * HSP] + gh[:, HSP:2 * HSP])
            n = jnp.tanh(gi[:, 2 * HSP:] + r * gh[:, 2 * HSP:])
            return (1.0 - z) * n + z * Hagg

        return step

    Hlast = jnp.zeros((B, HSP), f32)
    for lo, hi, ns in ((0, 9, 8), (9, 17, 16), (17, 25, 24), (25, N, N)):
        Hlast = jax.lax.fori_loop(lo, hi, _mkstep(ns), Hlast)
    out_ref[...] = jnp.dot(Hlast, wf_s[...],
                           preferred_element_type=f32) + sm_s[4:5, 0:NZP]


def kernel(x, adj, W_ih, W_hh, b_ih, b_hh, Wg, bg, Wm, Wf, bf):
    f32 = jnp.float32
    a_t = jnp.transpose(adj, (2, 1, 0)).astype(f32)

    out = pl.pallas_call(
        _dvae_body,
        out_shape=jax.ShapeDtypeStruct((B, NZP), f32),
        scratch_shapes=[
            pltpu.VMEM((N, B, HSP), f32),
            pltpu.VMEM((HSP, 3 * HSP), jnp.bfloat16),
            pltpu.VMEM((HSP, 2 * HSP), jnp.bfloat16),
            pltpu.VMEM((HSP, NZP), f32),
            pltpu.VMEM((N, 2 * HSP), f32),
            pltpu.VMEM((8, 3 * HSP), f32),
            pltpu.VMEM((N, B, 128), f32),
            pltpu.VMEM((N, B, 128), f32),
        ],
    )(x, adj, a_t, W_ih[:, 0][None], b_ih[None], b_hh[None], bg[None],
      bf[None], W_hh, Wg, Wm, Wf)
    return out[:, :NZ][:, :, None]


# speedup vs baseline: 1.4636x; 1.0378x over previous
import jax
import jax.numpy as jnp
from jax.experimental import pallas as pl
from jax.experimental.pallas import tpu as pltpu

B = 32
N = 32
HS = 501
NZ = 56
HSP = 512
NZP = 128


def _bmul(scal128, vec, nblk):
    return jnp.concatenate(
        [scal128 * vec[:, 128 * k:128 * (k + 1)] for k in range(nblk)], axis=1)


def _dvae_body(x_ref, adj_ref, at_ref, wi_r, bi_r, bh_r, bg_r, bf_r,
               whh_raw, wg_raw, wm_raw, wf_raw, out_ref,
               G, whh_s, wgm_s, wf_s, gme_s, sm_s, Xb_s, dcb_s):
    f32 = jnp.float32
    bf16 = jnp.bfloat16

    def _tpadded(blk):
        return jnp.transpose(
            jnp.pad(blk.astype(bf16), ((0, HSP - blk.shape[0]),
                                       (0, HSP - blk.shape[1]))))

    whh_s[:, 0:HSP] = _tpadded(whh_raw[0:HS, :])
    whh_s[:, HSP:2 * HSP] = _tpadded(whh_raw[HS:2 * HS, :])
    whh_s[:, 2 * HSP:3 * HSP] = _tpadded(whh_raw[2 * HS:3 * HS, :])
    wgm_s[:, 0:HSP] = _tpadded(wg_raw[:, 0:HS])
    wgm_s[:, HSP:2 * HSP] = _tpadded(wm_raw[:, 0:HS])
    wf_s[...] = jnp.transpose(
        jnp.pad(wf_raw[...], ((0, NZP - NZ), (0, HSP - HS))))
    gme_s[:, 0:HSP] = jnp.transpose(jnp.pad(wg_raw[:, HS:], ((0, HSP - HS),
                                                             (0, 0))))
    gme_s[:, HSP:] = jnp.transpose(jnp.pad(wm_raw[:, HS:], ((0, HSP - HS),
                                                            (0, 0))))
    sm_s[...] = jnp.zeros_like(sm_s)
    z11 = jnp.zeros((1, HSP - HS), f32)

    def _blocks(row):
        return jnp.concatenate([row[:, 0:HS], z11, row[:, HS:2 * HS], z11,
                                row[:, 2 * HS:3 * HS], z11], axis=1)

    sm_s[0:1, :] = _blocks(wi_r[...])
    sm_s[1:2, :] = _blocks(bi_r[...])
    sm_s[2:3, :] = _blocks(bh_r[...])
    sm_s[3:4, 0:HS] = bg_r[...]
    sm_s[4:5, 0:NZ] = bf_r[...]
    for w in range(N):
        Xb_s[w] = jnp.broadcast_to(x_ref[:, w:w + 1], (B, 128))
    wu_eq = (jax.lax.broadcasted_iota(jnp.int32, (N, N, 1), 0)
             == jax.lax.broadcasted_iota(jnp.int32, (N, N, 1), 1) + 1)
    dc_nb = jnp.sum(jnp.where(wu_eq, at_ref[...], 0.0), axis=1)
    dcb_s[...] = jnp.broadcast_to(dc_nb[:, :, None], (N, B, 128))

    G[...] = jnp.zeros_like(G)

    def _mkstep(ns):
        u_iota = jax.lax.broadcasted_iota(jnp.int32, (ns, B), 0)

        def step(w, Hprev):
            wm1 = jnp.maximum(w - 1, 0)
            coef = jnp.where(u_iota < w - 1,
                             at_ref[pl.ds(w, 1)][0][0:ns, :], 0.0)
            stable = jnp.sum(coef[:, :, None] * G[0:ns], axis=0)
            gm = (jnp.dot(Hprev.astype(bf16), wgm_s[...],
                          preferred_element_type=f32)
                  + gme_s[pl.ds(wm1, 1)] + sm_s[3:4, 0:2 * HSP])
            gated = jax.nn.sigmoid(gm[:, :HSP]) * gm[:, HSP:]
            G[pl.ds(wm1, 1)] = gated[None]
            Hagg = stable + _bmul(dcb_s[pl.ds(w, 1)][0], gated, 4)
            gi = _bmul(Xb_s[pl.ds(w, 1)][0], sm_s[0:1, :], 12) + sm_s[1:2, :]
            gh = (jnp.dot(Hagg.astype(bf16), whh_s[...],
                          preferred_element_type=f32) + sm_s[2:3, :])
            r = jax.nn.sigmoid(gi[:, :HSP] + gh[:, :HSP])
            z = jax.nn.sigmoid(gi[:, HSP:2 * HSP] + gh[:, HSP:2 * HSP])
            n = jnp.tanh(gi[:, 2 * HSP:] + r * gh[:, 2 * HSP:])
            return (1.0 - z) * n + z * Hagg

        return step

    Hlast = jnp.zeros((B, HSP), f32)
    for lo, hi, ns in ((0, 9, 8), (9, 17, 16), (17, 25, 24), (25, N, N)):
        Hlast = jax.lax.fori_loop(lo, hi, _mkstep(ns), Hlast, unroll=2)
    out_ref[...] = jnp.dot(Hlast, wf_s[...],
                           preferred_element_type=f32) + sm_s[4:5, 0:NZP]


def kernel(x, adj, W_ih, W_hh, b_ih, b_hh, Wg, bg, Wm, Wf, bf):
    f32 = jnp.float32
    a_t = jnp.transpose(adj, (2, 1, 0)).astype(f32)

    out = pl.pallas_call(
        _dvae_body,
        out_shape=jax.ShapeDtypeStruct((B, NZP), f32),
        scratch_shapes=[
            pltpu.VMEM((N, B, HSP), f32),
            pltpu.VMEM((HSP, 3 * HSP), jnp.bfloat16),
            pltpu.VMEM((HSP, 2 * HSP), jnp.bfloat16),
            pltpu.VMEM((HSP, NZP), f32),
            pltpu.VMEM((N, 2 * HSP), f32),
            pltpu.VMEM((8, 3 * HSP), f32),
            pltpu.VMEM((N, B, 128), f32),
            pltpu.VMEM((N, B, 128), f32),
        ],
    )(x, adj, a_t, W_ih[:, 0][None], b_ih[None], b_hh[None], bg[None],
      bf[None], W_hh, Wg, Wm, Wf)
    return out[:, :NZ][:, :, None]
